# Initial kernel scaffold; baseline (speedup 1.0000x reference)
import jax
import jax.numpy as jnp
from jax.experimental import pallas as pl


def kernel(x, W1, b1, g1, be1, W2, b2, g2, be2):
    raise NotImplementedError("write your pallas kernel here")


# speedup vs baseline: 918.1807x; 918.1807x over previous
import functools

import jax
import jax.numpy as jnp
import numpy as np
from jax.experimental import pallas as pl
from jax.experimental.pallas import tpu as pltpu

_T = 65536
_R = 512
_L = 128
_F32 = jnp.float32


def _ema_consts(a):
    af = np.float64(a)
    i = np.arange(_L)
    d = i[:, None] - i[None, :]
    m = np.where(d >= 0, af * (1.0 - af) ** np.maximum(d, 0), 0.0)
    q = (1.0 - af) ** _L
    k = np.arange(_R)
    dq = k[:, None] - k[None, :]
    qm = np.where(dq >= 0, q ** np.maximum(dq, 0), 0.0)
    dvec = (1.0 - af) ** (i + 1)
    qpow = q ** (k + 1)
    return (
        np.ascontiguousarray(m.T).astype(np.float32),
        qm.astype(np.float32),
        dvec.astype(np.float32).reshape(1, _L),
        qpow.astype(np.float32).reshape(_R, 1),
    )


_C_ESA = _ema_consts(2.0 / 11.0)
_C_WT1 = _ema_consts(2.0 / 12.0)
_C_WIL = _ema_consts(1.0 / 20.0)


def _shift_r(a, s):
    r, _ = a.shape
    up = jnp.concatenate([jnp.zeros((1, _L), _F32), a[: r - 1, :]], axis=0)
    return jnp.concatenate([up[:, _L - s:], a[:, : _L - s]], axis=1)


def _shift_l(a, s):
    dn = jnp.concatenate([a[1:, :], jnp.zeros((1, _L), _F32)], axis=0)
    return jnp.concatenate([a[:, s:], dn[:, :s]], axis=1)


_HI = jax.lax.Precision.HIGHEST


def _ema_block(x2d, mt, qm, dvec, qpow, x0):
    p = jnp.dot(x2d, mt, preferred_element_type=_F32, precision=_HI)
    e = p[:, _L - 1:_L]
    prev = jnp.dot(qm, e, preferred_element_type=_F32, precision=_HI) + qpow * x0
    prevex = jnp.concatenate([x0, prev[: _R - 1, :]], axis=0)
    return p + prevex * dvec


def _sma_win(v2d, w, row_i, lane_i, v0=None):
    acc = v2d
    for s in range(1, w):
        acc = acc + _shift_r(v2d, s)
    if v0 is not None:
        t = (row_i * _L + lane_i).astype(_F32)
        corr = jnp.maximum(jnp.float32(w - 1) - t, 0.0) * v0
        acc = acc + corr
    return acc * jnp.float32(1.0 / w)


def _make_xp(v2d, w, v0, row_i513, lane_i513):
    ext = jnp.concatenate([v2d, jnp.zeros((1, _L), _F32)], axis=0)
    b = _shift_r(ext, w - 1)
    fix = (row_i513 == 0) & (lane_i513 < w - 1)
    return jnp.where(fix, jnp.broadcast_to(v0, (_R + 1, _L)), b)


def _ladder(scr, width):
    for j in range(1, _L):
        scr[j:j + 1, :width] = scr[j:j + 1, :width] + scr[j - 1:j, :width]


def _replica_cumsums(xps, scr1, scr2):
    nb = len(xps)
    w1 = 513 * nb
    scr1[:, :w1] = jnp.concatenate([jnp.transpose(x) for x in xps], axis=1)
    _ladder(scr1, w1)
    w2 = 5 * nb
    t2 = []
    for b in range(nb):
        tot = scr1[_L - 1:_L, 513 * b:513 * (b + 1)]
        tot = jnp.pad(tot, ((0, 0), (0, 127)))
        t2.append(jnp.transpose(tot.reshape(5, _L)))
    scr2[:, :w2] = jnp.concatenate(t2, axis=1)
    _ladder(scr2, w2)
    outs = []
    for b in range(nb):
        p2 = scr2[:, 5 * b:5 * (b + 1)]
        tt = scr2[_L - 1:_L, 5 * b:5 * (b + 1)]
        s1 = tt[:, 0:1]
        s2 = s1 + tt[:, 1:2]
        s3 = s2 + tt[:, 2:3]
        s4 = s3 + tt[:, 3:4]
        excl2 = jnp.concatenate([jnp.zeros((1, 1), _F32), s1, s2, s3, s4], axis=1)
        c513 = p2 + excl2
        pref = jnp.transpose(c513).reshape(1, 640)[:, :513]
        exclrow = jnp.concatenate([jnp.zeros((1, 1), _F32), pref[:, :512]], axis=1)
        out_t = scr1[:, 513 * b:513 * (b + 1)] + exclrow
        outs.append(jnp.transpose(out_t))
    return outs


def _sma_from_cs(cs, w):
    d1 = _shift_l(cs, w - 1) if w > 1 else cs
    d2 = _shift_r(cs, 1)
    return (d1[:_R, :] - d2[:_R, :]) * jnp.float32(1.0 / w)


def _ind_kernel(close_ref, high_ref, low_ref,
                mt_e_ref, qm_e_ref, dv_e_ref, qp_e_ref,
                mt_t_ref, qm_t_ref, dv_t_ref, qp_t_ref,
                mt_w_ref, qm_w_ref, dv_w_ref, qp_w_ref,
                feats_ref, fmask_ref, scr1, scr2):
    close = close_ref[...]
    high = high_ref[...]
    low = low_ref[...]
    row = jax.lax.broadcasted_iota(jnp.int32, (_R, _L), 0)
    lane = jax.lax.broadcasted_iota(jnp.int32, (_R, _L), 1)
    row513 = jax.lax.broadcasted_iota(jnp.int32, (_R + 1, _L), 0)
    lane513 = jax.lax.broadcasted_iota(jnp.int32, (_R + 1, _L), 1)
    t0 = (row == 0) & (lane == 0)

    ema_e = functools.partial(_ema_block, mt=mt_e_ref[...], qm=qm_e_ref[...],
                              dvec=dv_e_ref[...], qpow=qp_e_ref[...])
    ema_t = functools.partial(_ema_block, mt=mt_t_ref[...], qm=qm_t_ref[...],
                              dvec=dv_t_ref[...], qpow=qp_t_ref[...])
    ema_w = functools.partial(_ema_block, mt=mt_w_ref[...], qm=qm_w_ref[...],
                              dvec=dv_w_ref[...], qpow=qp_w_ref[...])

    close_prev = _shift_r(close, 1)
    delta = jnp.where(t0, 0.0, close - close_prev)
    gain = jnp.maximum(delta, 0.0)
    loss = jnp.maximum(-delta, 0.0)

    def rsi(p):
        ag = _sma_win(gain, p, row, lane)
        al = _sma_win(loss, p, row, lane)
        rs = ag / (al + 1e-8)
        return 100.0 - 100.0 / (1.0 + rs)

    rsi14 = rsi(14)
    rsi9 = rsi(9)

    tp = (high + low + close) * jnp.float32(1.0 / 3.0)
    tp0 = tp[0:1, 0:1]
    esa = ema_e(tp, x0=tp0)
    ad = jnp.abs(tp - esa)
    dd = ema_e(ad, x0=ad[0:1, 0:1])
    ci = (tp - esa) / (0.015 * dd + 1e-8)
    wt1 = ema_t(ci, x0=ci[0:1, 0:1])
    wt2 = _sma_win(wt1, 4, row, lane, v0=wt1[0:1, 0:1])

    xp50 = _make_xp(close, 50, close[0:1, 0:1], row513, lane513)
    xp20t = _make_xp(tp, 20, tp0, row513, lane513)
    xp20r = _make_xp(delta, 20, delta[0:1, 0:1], row513, lane513)
    cs50, cs20t, cs20r = _replica_cumsums([xp50, xp20t, xp20r], scr1, scr2)
    sma50 = _sma_from_cs(cs50, 50)
    m20 = _sma_from_cs(cs20t, 20)
    rm = _sma_from_cs(cs20r, 20)

    madin = jnp.abs(tp - m20)
    sq = (delta - rm) * (delta - rm)
    xp20m = _make_xp(madin, 20, madin[0:1, 0:1], row513, lane513)
    xp20s = _make_xp(sq, 20, sq[0:1, 0:1], row513, lane513)
    cs20m, cs20s = _replica_cumsums([xp20m, xp20s], scr1, scr2)
    mad = _sma_from_cs(cs20m, 20)
    sqs = _sma_from_cs(cs20s, 20)

    cci = (tp - m20) / (0.015 * mad + 1e-8)

    high_prev = _shift_r(high, 1)
    low_prev = _shift_r(low, 1)
    up = jnp.where(t0, 0.0, high - high_prev)
    dn = jnp.where(t0, 0.0, low_prev - low)
    plus_dm = jnp.where((up > dn) & (up > 0.0), up, 0.0)
    minus_dm = jnp.where((dn > up) & (dn > 0.0), dn, 0.0)
    pc = jnp.where(t0, close, close_prev)
    tr = jnp.maximum(high - low, jnp.maximum(jnp.abs(high - pc), jnp.abs(low - pc)))
    trs = ema_w(tr, x0=tr[0:1, 0:1])
    pdm = ema_w(plus_dm, x0=plus_dm[0:1, 0:1])
    mdm = ema_w(minus_dm, x0=minus_dm[0:1, 0:1])
    pdi = 100.0 * pdm / (trs + 1e-8)
    mdi = 100.0 * mdm / (trs + 1e-8)
    dx = 100.0 * jnp.abs(pdi - mdi) / (pdi + mdi + 1e-8)
    adx = ema_w(dx, x0=dx[0:1, 0:1])

    vol = jnp.sqrt(sqs + 1e-8)
    volmean = jnp.sum(vol) * jnp.float32(1.0 / _T)
    vmask = jnp.where(vol > volmean, 1.0, 0.0)
    rmask = jnp.where(close > sma50, 1.0, 0.0)
    amask = jnp.where(adx > 20.0, 1.0, 0.0)
    fmask = vmask * rmask * amask

    feats_ref[0:1, :] = rsi14.reshape(1, _T)
    feats_ref[1:2, :] = wt1.reshape(1, _T)
    feats_ref[2:3, :] = wt2.reshape(1, _T)
    feats_ref[3:4, :] = cci.reshape(1, _T)
    feats_ref[4:5, :] = adx.reshape(1, _T)
    feats_ref[5:6, :] = rsi9.reshape(1, _T)
    feats_ref[6:8, :] = jnp.zeros((2, _T), _F32)
    fmask_ref[...] = fmask.reshape(1, _T)


_TILE = 8192
_K = _T // _TILE


def _mlp_kernel(f_ref, msk_ref, w1_ref, b1_ref, g1_ref, be1_ref,
                w2_ref, b2_ref, g2_ref, be2_ref,
                out_ref, acc1s, acc1q, acc2s, acc2q, carry):
    p = pl.program_id(0)
    k = pl.program_id(1)

    @pl.when((p == 0) & (k == 0))
    def _():
        acc1s[...] = jnp.zeros((64, _L), _F32)
        acc1q[...] = jnp.zeros((64, _L), _F32)
        acc2s[...] = jnp.zeros((32, _L), _F32)
        acc2q[...] = jnp.zeros((32, _L), _F32)

    ft = f_ref[...]
    z1 = jnp.dot(w1_ref[...], ft, preferred_element_type=_F32, precision=_HI) + b1_ref[...]

    @pl.when(p == 0)
    def _():
        acc1s[...] = acc1s[...] + jnp.sum(z1.reshape(64, _TILE // _L, _L), axis=1)
        acc1q[...] = acc1q[...] + jnp.sum((z1 * z1).reshape(64, _TILE // _L, _L), axis=1)

    inv_t = jnp.float32(1.0 / _T)
    m1 = jnp.sum(acc1s[...], axis=1, keepdims=True) * inv_t
    v1 = jnp.sum(acc1q[...], axis=1, keepdims=True) * inv_t - m1 * m1
    h = jnp.maximum(g1_ref[...] * (z1 - m1) / jnp.sqrt(v1 + 1e-5) + be1_ref[...], 0.0)
    z2 = jnp.dot(w2_ref[...], h, preferred_element_type=_F32, precision=_HI) + b2_ref[...]

    @pl.when(p == 1)
    def _():
        acc2s[...] = acc2s[...] + jnp.sum(z2.reshape(32, _TILE // _L, _L), axis=1)
        acc2q[...] = acc2q[...] + jnp.sum((z2 * z2).reshape(32, _TILE // _L, _L), axis=1)

    m2 = jnp.sum(acc2s[...], axis=1, keepdims=True) * inv_t
    v2 = jnp.sum(acc2q[...], axis=1, keepdims=True) * inv_t - m2 * m2
    ext = jnp.maximum(g2_ref[...] * (z2 - m2) / jnp.sqrt(v2 + 1e-5) + be2_ref[...], 0.0)
    comb = jnp.sum(ext, axis=0, keepdims=True) * jnp.float32(1.0 / 32.0)

    first = jnp.where(k == 0, comb[0:1, 0:1], carry[0:1, 0:1])
    shifted = jnp.concatenate([first, comb[:, : _TILE - 1]], axis=1)
    out_ref[...] = (comb + shifted) * 0.5 * msk_ref[...]
    carry[0:1, 0:1] = comb[0:1, _TILE - 1:_TILE]


def kernel(x, W1, b1, g1, be1, W2, b2, g2, be2):
    xt = x.astype(_F32)
    close = xt[:, 3].reshape(_R, _L)
    high = xt[:, 1].reshape(_R, _L)
    low = xt[:, 2].reshape(_R, _L)

    consts = []
    for c in (_C_ESA, _C_WT1, _C_WIL):
        consts.extend(jnp.asarray(a) for a in c)

    feats, fmask = pl.pallas_call(
        _ind_kernel,
        out_shape=[
            jax.ShapeDtypeStruct((8, _T), _F32),
            jax.ShapeDtypeStruct((1, _T), _F32),
        ],
        scratch_shapes=[
            pltpu.VMEM((_L, 513 * 3), _F32),
            pltpu.VMEM((_L, 16), _F32),
        ],
    )(close, high, low, *consts)

    w1p = jnp.pad(W1.astype(_F32), ((0, 0), (0, 2)))
    out = pl.pallas_call(
        _mlp_kernel,
        grid=(3, _K),
        in_specs=[
            pl.BlockSpec((8, _TILE), lambda p, k: (0, k)),
            pl.BlockSpec((1, _TILE), lambda p, k: (0, k)),
            pl.BlockSpec((64, 8), lambda p, k: (0, 0)),
            pl.BlockSpec((64, 1), lambda p, k: (0, 0)),
            pl.BlockSpec((64, 1), lambda p, k: (0, 0)),
            pl.BlockSpec((64, 1), lambda p, k: (0, 0)),
            pl.BlockSpec((32, 64), lambda p, k: (0, 0)),
            pl.BlockSpec((32, 1), lambda p, k: (0, 0)),
            pl.BlockSpec((32, 1), lambda p, k: (0, 0)),
            pl.BlockSpec((32, 1), lambda p, k: (0, 0)),
        ],
        out_specs=pl.BlockSpec((1, _TILE), lambda p, k: (0, k)),
        out_shape=jax.ShapeDtypeStruct((1, _T), _F32),
        scratch_shapes=[
            pltpu.VMEM((64, _L), _F32),
            pltpu.VMEM((64, _L), _F32),
            pltpu.VMEM((32, _L), _F32),
            pltpu.VMEM((32, _L), _F32),
            pltpu.VMEM((1, 1), _F32),
        ],
    )(feats, fmask, w1p,
      b1.astype(_F32).reshape(64, 1), g1.astype(_F32).reshape(64, 1),
      be1.astype(_F32).reshape(64, 1), W2.astype(_F32),
      b2.astype(_F32).reshape(32, 1), g2.astype(_F32).reshape(32, 1),
      be2.astype(_F32).reshape(32, 1))
    return out.reshape(_T)


# speedup vs baseline: 1745.6931x; 1.9013x over previous
import functools

import jax
import jax.numpy as jnp
import numpy as np
from jax.experimental import pallas as pl
from jax.experimental.pallas import tpu as pltpu

_T = 65536
_R = 512
_L = 128
_F32 = jnp.float32


def _ema_consts(a):
    af = np.float64(a)
    i = np.arange(_L)
    d = i[:, None] - i[None, :]
    m = np.where(d >= 0, af * (1.0 - af) ** np.maximum(d, 0), 0.0)
    q = (1.0 - af) ** _L
    k = np.arange(_R)
    dq = k[:, None] - k[None, :]
    qm = np.where(dq >= 0, q ** np.maximum(dq, 0), 0.0)
    dvec = (1.0 - af) ** (i + 1)
    qpow = q ** (k + 1)
    return (
        np.ascontiguousarray(m.T).astype(np.float32),
        qm.astype(np.float32),
        dvec.astype(np.float32).reshape(1, _L),
        qpow.astype(np.float32).reshape(_R, 1),
    )


_C_ESA = _ema_consts(2.0 / 11.0)
_C_WT1 = _ema_consts(2.0 / 12.0)
_C_WIL = _ema_consts(1.0 / 20.0)


def _shift_r(a, s):
    r, _ = a.shape
    up = jnp.concatenate([jnp.zeros((1, _L), _F32), a[: r - 1, :]], axis=0)
    return jnp.concatenate([up[:, _L - s:], a[:, : _L - s]], axis=1)


def _shift_l(a, s):
    dn = jnp.concatenate([a[1:, :], jnp.zeros((1, _L), _F32)], axis=0)
    return jnp.concatenate([a[:, s:], dn[:, :s]], axis=1)


_HI = jax.lax.Precision.HIGHEST


def _ema_block(x2d, mt, qm, dvec, qpow, x0):
    p = jnp.dot(x2d, mt, preferred_element_type=_F32, precision=_HI)
    e = p[:, _L - 1:_L]
    prev = jnp.dot(qm, e, preferred_element_type=_F32, precision=_HI) + qpow * x0
    prevex = jnp.concatenate([x0, prev[: _R - 1, :]], axis=0)
    return p + prevex * dvec


def _sma_win(v2d, w, row_i, lane_i, v0=None):
    acc = v2d
    for s in range(1, w):
        acc = acc + _shift_r(v2d, s)
    if v0 is not None:
        t = (row_i * _L + lane_i).astype(_F32)
        corr = jnp.maximum(jnp.float32(w - 1) - t, 0.0) * v0
        acc = acc + corr
    return acc * jnp.float32(1.0 / w)


def _make_xp(v2d, w, v0, row_i513, lane_i513):
    ext = jnp.concatenate([v2d, jnp.zeros((1, _L), _F32)], axis=0)
    b = _shift_r(ext, w - 1)
    fix = (row_i513 == 0) & (lane_i513 < w - 1)
    return jnp.where(fix, jnp.broadcast_to(v0, (_R + 1, _L)), b)


def _ladder(scr, width):
    for j in range(1, _L):
        scr[j:j + 1, :width] = scr[j:j + 1, :width] + scr[j - 1:j, :width]


def _replica_cumsums(xps, scr1, scr2):
    nb = len(xps)
    w1 = 513 * nb
    scr1[:, :w1] = jnp.concatenate([jnp.transpose(x) for x in xps], axis=1)
    _ladder(scr1, w1)
    w2 = 5 * nb
    t2 = []
    for b in range(nb):
        tot = scr1[_L - 1:_L, 513 * b:513 * (b + 1)]
        tot = jnp.pad(tot, ((0, 0), (0, 127)))
        t2.append(jnp.transpose(tot.reshape(5, _L)))
    scr2[:, :w2] = jnp.concatenate(t2, axis=1)
    _ladder(scr2, w2)
    outs = []
    for b in range(nb):
        p2 = scr2[:, 5 * b:5 * (b + 1)]
        tt = scr2[_L - 1:_L, 5 * b:5 * (b + 1)]
        s1 = tt[:, 0:1]
        s2 = s1 + tt[:, 1:2]
        s3 = s2 + tt[:, 2:3]
        s4 = s3 + tt[:, 3:4]
        excl2 = jnp.concatenate([jnp.zeros((1, 1), _F32), s1, s2, s3, s4], axis=1)
        c513 = p2 + excl2
        pref = jnp.transpose(c513).reshape(1, 640)[:, :513]
        exclrow = jnp.concatenate([jnp.zeros((1, 1), _F32), pref[:, :512]], axis=1)
        out_t = scr1[:, 513 * b:513 * (b + 1)] + exclrow
        outs.append(jnp.transpose(out_t))
    return outs


def _sma_from_cs(cs, w):
    d1 = _shift_l(cs, w - 1) if w > 1 else cs
    d2 = _shift_r(cs, 1)
    return (d1[:_R, :] - d2[:_R, :]) * jnp.float32(1.0 / w)


def _ind_kernel(close_ref, high_ref, low_ref,
                mt_e_ref, qm_e_ref, dv_e_ref, qp_e_ref,
                mt_t_ref, qm_t_ref, dv_t_ref, qp_t_ref,
                mt_w_ref, qm_w_ref, dv_w_ref, qp_w_ref,
                feats_ref, fmask_ref, mom_ref, scr1, scr2):
    close = close_ref[...]
    high = high_ref[...]
    low = low_ref[...]
    row = jax.lax.broadcasted_iota(jnp.int32, (_R, _L), 0)
    lane = jax.lax.broadcasted_iota(jnp.int32, (_R, _L), 1)
    row513 = jax.lax.broadcasted_iota(jnp.int32, (_R + 1, _L), 0)
    lane513 = jax.lax.broadcasted_iota(jnp.int32, (_R + 1, _L), 1)
    t0 = (row == 0) & (lane == 0)

    ema_e = functools.partial(_ema_block, mt=mt_e_ref[...], qm=qm_e_ref[...],
                              dvec=dv_e_ref[...], qpow=qp_e_ref[...])
    ema_t = functools.partial(_ema_block, mt=mt_t_ref[...], qm=qm_t_ref[...],
                              dvec=dv_t_ref[...], qpow=qp_t_ref[...])
    ema_w = functools.partial(_ema_block, mt=mt_w_ref[...], qm=qm_w_ref[...],
                              dvec=dv_w_ref[...], qpow=qp_w_ref[...])

    close_prev = _shift_r(close, 1)
    delta = jnp.where(t0, 0.0, close - close_prev)
    gain = jnp.maximum(delta, 0.0)
    loss = jnp.maximum(-delta, 0.0)

    def rsi(p):
        ag = _sma_win(gain, p, row, lane)
        al = _sma_win(loss, p, row, lane)
        rs = ag / (al + 1e-8)
        return 100.0 - 100.0 / (1.0 + rs)

    rsi14 = rsi(14)
    rsi9 = rsi(9)

    tp = (high + low + close) * jnp.float32(1.0 / 3.0)
    tp0 = tp[0:1, 0:1]
    esa = ema_e(tp, x0=tp0)
    ad = jnp.abs(tp - esa)
    dd = ema_e(ad, x0=ad[0:1, 0:1])
    ci = (tp - esa) / (0.015 * dd + 1e-8)
    wt1 = ema_t(ci, x0=ci[0:1, 0:1])
    wt2 = _sma_win(wt1, 4, row, lane, v0=wt1[0:1, 0:1])

    xp50 = _make_xp(close, 50, close[0:1, 0:1], row513, lane513)
    xp20t = _make_xp(tp, 20, tp0, row513, lane513)
    xp20r = _make_xp(delta, 20, delta[0:1, 0:1], row513, lane513)
    cs50, cs20t, cs20r = _replica_cumsums([xp50, xp20t, xp20r], scr1, scr2)
    sma50 = _sma_from_cs(cs50, 50)
    m20 = _sma_from_cs(cs20t, 20)
    rm = _sma_from_cs(cs20r, 20)

    madin = jnp.abs(tp - m20)
    sq = (delta - rm) * (delta - rm)
    xp20m = _make_xp(madin, 20, madin[0:1, 0:1], row513, lane513)
    xp20s = _make_xp(sq, 20, sq[0:1, 0:1], row513, lane513)
    cs20m, cs20s = _replica_cumsums([xp20m, xp20s], scr1, scr2)
    mad = _sma_from_cs(cs20m, 20)
    sqs = _sma_from_cs(cs20s, 20)

    cci = (tp - m20) / (0.015 * mad + 1e-8)

    high_prev = _shift_r(high, 1)
    low_prev = _shift_r(low, 1)
    up = jnp.where(t0, 0.0, high - high_prev)
    dn = jnp.where(t0, 0.0, low_prev - low)
    plus_dm = jnp.where((up > dn) & (up > 0.0), up, 0.0)
    minus_dm = jnp.where((dn > up) & (dn > 0.0), dn, 0.0)
    pc = jnp.where(t0, close, close_prev)
    tr = jnp.maximum(high - low, jnp.maximum(jnp.abs(high - pc), jnp.abs(low - pc)))
    trs = ema_w(tr, x0=tr[0:1, 0:1])
    pdm = ema_w(plus_dm, x0=plus_dm[0:1, 0:1])
    mdm = ema_w(minus_dm, x0=minus_dm[0:1, 0:1])
    pdi = 100.0 * pdm / (trs + 1e-8)
    mdi = 100.0 * mdm / (trs + 1e-8)
    dx = 100.0 * jnp.abs(pdi - mdi) / (pdi + mdi + 1e-8)
    adx = ema_w(dx, x0=dx[0:1, 0:1])

    vol = jnp.sqrt(sqs + 1e-8)
    volmean = jnp.sum(vol) * jnp.float32(1.0 / _T)
    vmask = jnp.where(vol > volmean, 1.0, 0.0)
    rmask = jnp.where(close > sma50, 1.0, 0.0)
    amask = jnp.where(adx > 20.0, 1.0, 0.0)
    fmask = vmask * rmask * amask

    f6 = jnp.concatenate(
        [v.reshape(1, _T) for v in (rsi14, wt1, wt2, cci, adx, rsi9)], axis=0)
    feats_ref[0:6, :] = f6
    feats_ref[6:8, :] = jnp.zeros((2, _T), _F32)
    fmask_ref[...] = fmask.reshape(1, _T)

    gmat = jax.lax.dot_general(f6, f6, (((1,), (1,)), ((), ())),
                               preferred_element_type=_F32, precision=_HI)
    sf = jnp.sum(f6, axis=1, keepdims=True)
    mom_ref[...] = jnp.pad(
        jnp.concatenate([jnp.pad(gmat, ((0, 0), (0, 2))), sf], axis=1),
        ((0, 2), (0, _L - 9)))


_TILE = 8192
_K = _T // _TILE


def _mlp_kernel(f_ref, msk_ref, mom_ref, w1_ref, b1_ref, g1_ref, be1_ref,
                w2_ref, b2_ref, g2_ref, be2_ref,
                out_ref, z2s, acc2s, acc2q, carry):
    p = pl.program_id(0)
    k = pl.program_id(1)
    inv_t = jnp.float32(1.0 / _T)

    @pl.when((p == 0) & (k == 0))
    def _():
        acc2s[...] = jnp.zeros((32, _L), _F32)
        acc2q[...] = jnp.zeros((32, _L), _F32)

    @pl.when(p == 0)
    def _():
        w1 = w1_ref[...]
        b1 = b1_ref[...]
        umean = jnp.dot(w1, mom_ref[0:8, 8:9],
                        preferred_element_type=_F32, precision=_HI) * inv_t
        e2 = jnp.sum(jnp.dot(w1, mom_ref[0:8, 0:8],
                             preferred_element_type=_F32, precision=_HI) * w1,
                     axis=1, keepdims=True) * inv_t
        m1 = umean + b1
        v1 = e2 - umean * umean
        z1 = jnp.dot(w1, f_ref[...], preferred_element_type=_F32,
                     precision=_HI) + b1
        h = jnp.maximum(
            g1_ref[...] * (z1 - m1) / jnp.sqrt(v1 + 1e-5) + be1_ref[...], 0.0)
        z2 = jnp.dot(w2_ref[...], h, preferred_element_type=_F32,
                     precision=_HI) + b2_ref[...]
        z2s[:, pl.ds(k * _TILE, _TILE)] = z2
        acc2s[...] = acc2s[...] + jnp.sum(z2.reshape(32, _TILE // _L, _L), axis=1)
        acc2q[...] = acc2q[...] + jnp.sum((z2 * z2).reshape(32, _TILE // _L, _L), axis=1)

    @pl.when(p == 1)
    def _():
        z2 = z2s[:, pl.ds(k * _TILE, _TILE)]
        m2 = jnp.sum(acc2s[...], axis=1, keepdims=True) * inv_t
        v2 = jnp.sum(acc2q[...], axis=1, keepdims=True) * inv_t - m2 * m2
        ext = jnp.maximum(
            g2_ref[...] * (z2 - m2) / jnp.sqrt(v2 + 1e-5) + be2_ref[...], 0.0)
        comb = jnp.sum(ext, axis=0, keepdims=True) * jnp.float32(1.0 / 32.0)

        first = jnp.where(k == 0, comb[0:1, 0:1], carry[0:1, 0:1])
        shifted = jnp.concatenate([first, comb[:, : _TILE - 1]], axis=1)
        out_ref[...] = (comb + shifted) * 0.5 * msk_ref[...]
        carry[0:1, 0:1] = comb[0:1, _TILE - 1:_TILE]


def kernel(x, W1, b1, g1, be1, W2, b2, g2, be2):
    xt = x.astype(_F32)
    close = xt[:, 3].reshape(_R, _L)
    high = xt[:, 1].reshape(_R, _L)
    low = xt[:, 2].reshape(_R, _L)

    consts = []
    for c in (_C_ESA, _C_WT1, _C_WIL):
        consts.extend(jnp.asarray(a) for a in c)

    feats, fmask, mom = pl.pallas_call(
        _ind_kernel,
        out_shape=[
            jax.ShapeDtypeStruct((8, _T), _F32),
            jax.ShapeDtypeStruct((1, _T), _F32),
            jax.ShapeDtypeStruct((8, _L), _F32),
        ],
        scratch_shapes=[
            pltpu.VMEM((_L, 513 * 3), _F32),
            pltpu.VMEM((_L, 16), _F32),
        ],
    )(close, high, low, *consts)

    w1p = jnp.pad(W1.astype(_F32), ((0, 0), (0, 2)))
    out = pl.pallas_call(
        _mlp_kernel,
        grid=(2, _K),
        in_specs=[
            pl.BlockSpec((8, _TILE), lambda p, k: (0, k)),
            pl.BlockSpec((1, _TILE), lambda p, k: (0, k)),
            pl.BlockSpec((8, _L), lambda p, k: (0, 0)),
            pl.BlockSpec((64, 8), lambda p, k: (0, 0)),
            pl.BlockSpec((64, 1), lambda p, k: (0, 0)),
            pl.BlockSpec((64, 1), lambda p, k: (0, 0)),
            pl.BlockSpec((64, 1), lambda p, k: (0, 0)),
            pl.BlockSpec((32, 64), lambda p, k: (0, 0)),
            pl.BlockSpec((32, 1), lambda p, k: (0, 0)),
            pl.BlockSpec((32, 1), lambda p, k: (0, 0)),
            pl.BlockSpec((32, 1), lambda p, k: (0, 0)),
        ],
        out_specs=pl.BlockSpec((1, _TILE), lambda p, k: (0, k)),
        out_shape=jax.ShapeDtypeStruct((1, _T), _F32),
        scratch_shapes=[
            pltpu.VMEM((32, _T), _F32),
            pltpu.VMEM((32, _L), _F32),
            pltpu.VMEM((32, _L), _F32),
            pltpu.VMEM((1, 1), _F32),
        ],
    )(feats, fmask, mom, w1p,
      b1.astype(_F32).reshape(64, 1), g1.astype(_F32).reshape(64, 1),
      be1.astype(_F32).reshape(64, 1), W2.astype(_F32),
      b2.astype(_F32).reshape(32, 1), g2.astype(_F32).reshape(32, 1),
      be2.astype(_F32).reshape(32, 1))
    return out.reshape(_T)
